# baseline (speedup 1.0000x reference)
import functools
import math

import jax
import jax.numpy as jnp
from jax import lax
from jax.experimental import pallas as pl
from jax.experimental.pallas import tpu as pltpu
from jax.experimental.pallas import tpu_sc as plsc

D = 128
LANES = 16
NC = 2
NS = 16
NW = NC * NS
NBUF = 5


def _build(batch, hist):
    per_w = batch // NW
    assert per_w * NW == batch and per_w <= 128
    assert hist % NBUF == 0
    scale = math.sqrt(float(D))

    mesh = plsc.VectorSubcoreMesh(core_axis_name="c", subcore_axis_name="s")

    @functools.partial(
        pl.kernel,
        out_type=jax.ShapeDtypeStruct((hist, batch, D), jnp.float32),
        mesh=mesh,
        compiler_params=pltpu.CompilerParams(skip_device_barrier=True),
        scratch_types=[
            pltpu.VMEM((hist, per_w), jnp.int32),
            [pltpu.VMEM((per_w, D), jnp.float32) for _ in range(NBUF)],
            [pltpu.SemaphoreType.DMA for _ in range(NBUF)],
            [pltpu.SemaphoreType.DMA for _ in range(NBUF)],
        ],
    )
    def emb_kernel(idx_hbm, table_hbm, out_hbm, idx_v, bufs, gsems, ssems):
        wid = lax.axis_index("s") * NC + lax.axis_index("c")
        i0 = wid * per_w

        pltpu.sync_copy(idx_hbm.at[:, pl.ds(i0, per_w)], idx_v)

        def fire_gather(g, b):
            pltpu.async_copy(table_hbm.at[idx_v.at[g]], bufs[b], gsems[b])

        def wait_gather(g, b):
            pltpu.make_async_copy(
                table_hbm.at[idx_v.at[g]], bufs[b], gsems[b]).wait()

        def out_slice(g):
            return out_hbm.at[g, pl.ds(i0, per_w)]

        def fire_scatter(g, b):
            pltpu.async_copy(bufs[b], out_slice(g), ssems[b])

        def wait_scatter(g, b):
            pltpu.make_async_copy(bufs[b], out_slice(g), ssems[b]).wait()

        def scale_buf(b):
            buf = bufs[b]

            def row(r, carry):
                for c in range(D // LANES):
                    sl = (r, pl.ds(c * LANES, LANES))
                    buf[sl] = buf[sl] * scale
                return carry

            lax.fori_loop(0, per_w, row, 0)

        for b in range(NBUF):
            fire_gather(b, b)

        @pl.loop(0, hist, step=NBUF)
        def _(go):
            for b in range(NBUF):
                g = go + b
                wait_gather(g, b)
                scale_buf(b)
                fire_scatter(g, b)

            @pl.when(go < hist - NBUF)
            def _():
                for b in range(NBUF):
                    g = go + b
                    wait_scatter(g, b)
                    fire_gather(g + NBUF, b)

        for b in range(NBUF):
            wait_scatter(hist - NBUF + b, b)

    return emb_kernel


def kernel(indices, table):
    batch, hist = indices.shape
    out = _build(batch, hist)(indices.T, table)
    return out.transpose(1, 0, 2)


# speedup vs baseline: 1.0153x; 1.0153x over previous
import functools
import math

import jax
import jax.numpy as jnp
from jax import lax
from jax.experimental import pallas as pl
from jax.experimental.pallas import tpu as pltpu
from jax.experimental.pallas import tpu_sc as plsc

D = 128
LANES = 16
NC = 2
NS = 16
NW = NC * NS
NBUF = 5


def _build(batch, hist):
    per_w = batch // NW
    assert per_w * NW == batch and per_w <= 128
    assert hist % NBUF == 0
    scale = math.sqrt(float(D))

    mesh = plsc.VectorSubcoreMesh(core_axis_name="c", subcore_axis_name="s")

    @functools.partial(
        pl.kernel,
        out_type=jax.ShapeDtypeStruct((hist, batch, D), jnp.float32),
        mesh=mesh,
        scratch_types=[
            pltpu.VMEM((hist, per_w), jnp.int32),
            [pltpu.VMEM((per_w, D), jnp.float32) for _ in range(NBUF)],
            [pltpu.SemaphoreType.DMA for _ in range(NBUF)],
            [pltpu.SemaphoreType.DMA for _ in range(NBUF)],
        ],
    )
    def emb_kernel(idx_hbm, table_hbm, out_hbm, idx_v, bufs, gsems, ssems):
        wid = lax.axis_index("s") * NC + lax.axis_index("c")
        i0 = wid * per_w

        pltpu.sync_copy(idx_hbm.at[:, pl.ds(i0, per_w)], idx_v)

        def fire_gather(g, b):
            pltpu.async_copy(table_hbm.at[idx_v.at[g]], bufs[b], gsems[b])

        def wait_gather(g, b):
            pltpu.make_async_copy(
                table_hbm.at[idx_v.at[g]], bufs[b], gsems[b]).wait()

        def out_slice(g):
            return out_hbm.at[g, pl.ds(i0, per_w)]

        def fire_scatter(g, b):
            pltpu.async_copy(bufs[b], out_slice(g), ssems[b])

        def wait_scatter(g, b):
            pltpu.make_async_copy(bufs[b], out_slice(g), ssems[b]).wait()

        def scale_buf(b):
            buf = bufs[b]

            def row(r, carry):
                for c in range(D // LANES):
                    sl = (r, pl.ds(c * LANES, LANES))
                    buf[sl] = buf[sl] * scale
                return carry

            lax.fori_loop(0, per_w, row, 0)

        for b in range(NBUF):
            fire_gather(b, b)

        @pl.loop(0, hist, step=NBUF)
        def _(go):
            for b in range(NBUF):
                g = go + b
                wait_gather(g, b)
                scale_buf(b)
                fire_scatter(g, b)

                b2 = (b - 2) % NBUF
                g2 = g - 2

                @pl.when(jnp.logical_and(g2 >= 0, g2 + NBUF < hist))
                def _():
                    wait_scatter(g2, b2)
                    fire_gather(g2 + NBUF, b2)

        for b in range(NBUF):
            wait_scatter(hist - NBUF + b, b)

    return emb_kernel


def kernel(indices, table):
    batch, hist = indices.shape
    out = _build(batch, hist)(indices.T, table)
    return out.transpose(1, 0, 2)


# speedup vs baseline: 1.0336x; 1.0180x over previous
import functools
import math

import jax
import jax.numpy as jnp
from jax import lax
from jax.experimental import pallas as pl
from jax.experimental.pallas import tpu as pltpu
from jax.experimental.pallas import tpu_sc as plsc

D = 128
LANES = 16
NC = 2
NS = 16
NW = NC * NS
NBUF = 10
HALF = 64


def _build(batch, hist):
    per_w = batch // NW
    assert per_w * NW == batch and per_w == 2 * HALF
    n_chunks = 2 * hist
    assert n_chunks % NBUF == 0
    scale = math.sqrt(float(D))

    mesh = plsc.VectorSubcoreMesh(core_axis_name="c", subcore_axis_name="s")

    @functools.partial(
        pl.kernel,
        out_type=jax.ShapeDtypeStruct((hist, batch, D), jnp.float32),
        mesh=mesh,
        scratch_types=[
            pltpu.VMEM((hist, per_w), jnp.int32),
            [pltpu.VMEM((HALF, D), jnp.float32) for _ in range(NBUF)],
            [pltpu.SemaphoreType.DMA for _ in range(NBUF)],
            [pltpu.SemaphoreType.DMA for _ in range(NBUF)],
        ],
    )
    def emb_kernel(idx_hbm, table_hbm, out_hbm, idx_v, bufs, gsems, ssems):
        wid = lax.axis_index("s") * NC + lax.axis_index("c")
        i0 = wid * per_w

        pltpu.sync_copy(idx_hbm.at[:, pl.ds(i0, per_w)], idx_v)

        def idx_slice(g, h):
            return idx_v.at[g // 2, pl.ds(h * HALF, HALF)]

        def fire_gather(g, h, b):
            pltpu.async_copy(table_hbm.at[idx_slice(g, h)], bufs[b], gsems[b])

        def wait_gather(g, h, b):
            pltpu.make_async_copy(
                table_hbm.at[idx_slice(g, h)], bufs[b], gsems[b]).wait()

        def out_slice(g, h):
            return out_hbm.at[g // 2, pl.ds(i0 + h * HALF, HALF)]

        def fire_scatter(g, h, b):
            pltpu.async_copy(bufs[b], out_slice(g, h), ssems[b])

        def wait_scatter(g, h, b):
            pltpu.make_async_copy(
                bufs[b], out_slice(g, h), ssems[b]).wait()

        def scale_buf(b):
            buf = bufs[b]

            def row(r, carry):
                for c in range(D // LANES):
                    sl = (r, pl.ds(c * LANES, LANES))
                    buf[sl] = buf[sl] * scale
                return carry

            lax.fori_loop(0, HALF, row, 0)

        for b in range(NBUF):
            fire_gather(b, b % 2, b)

        @pl.loop(0, n_chunks, step=NBUF)
        def _(go):
            for b in range(NBUF):
                g = go + b
                h = b % 2
                wait_gather(g, h, b)
                scale_buf(b)
                fire_scatter(g, h, b)

                b2 = (b - 2) % NBUF
                g2 = g - 2

                @pl.when(jnp.logical_and(g2 >= 0, g2 + NBUF < n_chunks))
                def _():
                    wait_scatter(g2, h, b2)
                    fire_gather(g2 + NBUF, h, b2)

        for b in range(NBUF):
            wait_scatter(n_chunks - NBUF + b, b % 2, b)

    return emb_kernel


def kernel(indices, table):
    batch, hist = indices.shape
    out = _build(batch, hist)(indices.T, table)
    return out.transpose(1, 0, 2)
